# baseline (speedup 1.0000x reference)
import jax
import jax.numpy as jnp
from jax.experimental import pallas as pl
from jax.experimental.pallas import tpu as pltpu

T = 2048
D = 2048
L = 32768
K = 64

_BL_ENC = 512
_BK_DEC = 512


def _enc_body(x_ref, w_ref, b_ref, o_ref):
    acc = jax.lax.dot_general(
        x_ref[...], w_ref[...], (((1,), (1,)), ((), ())),
        preferred_element_type=jnp.float32)
    o_ref[...] = jnp.maximum(acc + b_ref[...], 0.0)


def _dec_body(z_ref, w_ref, b_ref, o_ref):
    k = pl.program_id(0)

    @pl.when(k == 0)
    def _init():
        o_ref[...] = jnp.broadcast_to(b_ref[...], o_ref.shape)

    o_ref[...] += jax.lax.dot_general(
        z_ref[...], w_ref[...], (((1,), (0,)), ((), ())),
        preferred_element_type=jnp.float32)


def kernel(x, W_enc, b_enc, W_dec, b_dec):
    sae_in = x - b_dec[None, :]

    pre_acts = pl.pallas_call(
        _enc_body,
        grid=(L // _BL_ENC,),
        in_specs=[
            pl.BlockSpec((T, D), lambda j: (0, 0)),
            pl.BlockSpec((_BL_ENC, D), lambda j: (j, 0)),
            pl.BlockSpec((1, _BL_ENC), lambda j: (0, j)),
        ],
        out_specs=pl.BlockSpec((T, _BL_ENC), lambda j: (0, j)),
        out_shape=jax.ShapeDtypeStruct((T, L), jnp.float32),
    )(sae_in, W_enc, b_enc.reshape(1, L))

    top_acts, top_indices = jax.lax.top_k(pre_acts, K)

    rows = jnp.arange(T)[:, None]
    z = jnp.zeros((T, L), dtype=x.dtype).at[rows, top_indices].add(top_acts)

    sae_out = pl.pallas_call(
        _dec_body,
        grid=(L // _BK_DEC,),
        in_specs=[
            pl.BlockSpec((T, _BK_DEC), lambda k: (0, k)),
            pl.BlockSpec((_BK_DEC, D), lambda k: (k, 0)),
            pl.BlockSpec((1, D), lambda k: (0, 0)),
        ],
        out_specs=pl.BlockSpec((T, D), lambda k: (0, 0)),
        out_shape=jax.ShapeDtypeStruct((T, D), jnp.float32),
    )(z, W_dec, b_dec.reshape(1, D))

    e = sae_out - x
    total_variance = jnp.sum((x - jnp.mean(x, axis=0)) ** 2)
    fvu = jnp.sum(e * e) / total_variance
    auxk_loss = jnp.zeros(())
    multi_topk_fvu = jnp.zeros(())
    return sae_out, top_acts, top_indices, fvu, auxk_loss, multi_topk_fvu


# speedup vs baseline: 7.7098x; 7.7098x over previous
import functools

import jax
import jax.numpy as jnp
from jax import lax
from jax.experimental import pallas as pl
from jax.experimental.pallas import tpu as pltpu
from jax.experimental.pallas import tpu_sc as plsc

T = 2048
D = 2048
L = 32768
K = 64

NC = 2
NS = 16
NW = NC * NS
TPW = T // NW
NSUB = L // 16
NGRP = NSUB // 16
CAND_CAP = 1024
NEG = -3.0e38

_BL_ENC = 512
_BK_DEC = 512


def _enc_body(x_ref, w_ref, b_ref, o_ref):
    acc = jax.lax.dot_general(
        x_ref[...], w_ref[...], (((1,), (1,)), ((), ())),
        preferred_element_type=jnp.float32)
    o_ref[...] = jnp.maximum(acc + b_ref[...], 0.0)


def _dec_body(z_ref, w_ref, b_ref, o_ref):
    k = pl.program_id(0)

    @pl.when(k == 0)
    def _init():
        o_ref[...] = jnp.broadcast_to(b_ref[...], o_ref.shape)

    o_ref[...] += jax.lax.dot_general(
        z_ref[...], w_ref[...], (((1,), (0,)), ((), ())),
        preferred_element_type=jnp.float32)


def _merge2(ka, va, kb, vb):
    kbr = lax.rev(kb, (0,))
    vbr = lax.rev(vb, (0,))
    m = ka >= kbr
    khi = jnp.where(m, ka, kbr)
    vhi = jnp.where(m, va, vbr)
    klo = jnp.where(m, kbr, ka)
    vlo = jnp.where(m, vbr, va)
    khi, vhi = plsc.sort_key_val(khi, vhi, descending=True)
    klo, vlo = plsc.sort_key_val(klo, vlo, descending=True)
    return khi, vhi, klo, vlo


def _cascade_insert(R, kv, vv):
    ck, cv = plsc.sort_key_val(kv, vv, descending=True)
    out = []
    for i in range(4):
        khi, vhi, ck, cv = _merge2(R[2 * i], R[2 * i + 1], ck, cv)
        out.append(khi)
        out.append(vhi)
    return tuple(out)


def _maybe_insert(R, kv, vv, gate_splat):
    pc = plsc.all_reduce_population_count(kv >= gate_splat)
    return lax.cond(pc[0] > 0,
                    lambda: _cascade_insert(R, kv, vv),
                    lambda: R)


def _topk_body(pre_hbm, acts_hbm, idx_hbm,
               databuf, bmbuf, subbuf, candv, candi, stage_a, stage_i):
    cid = lax.axis_index("c")
    sid = lax.axis_index("s")
    wid = sid * NC + cid
    t0 = wid * TPW
    iota = lax.iota(jnp.int32, 16)
    neg_splat = jnp.full((16,), NEG, jnp.float32)
    zero_i = jnp.zeros((16,), jnp.int32)

    def token_step(tt, _):
        pltpu.sync_copy(pre_hbm.at[t0 + tt], databuf)

        def ph_a(g, _):
            base = g * 256
            m = databuf[pl.ds(base, 16)]
            for i in range(1, 16):
                m = jnp.maximum(m, databuf[pl.ds(base + i * 16, 16)])
            bmbuf[pl.ds(g * 16, 16)] = m
            return 0
        lax.fori_loop(0, NGRP, ph_a, 0, unroll=2)

        bm2 = []
        for i in range(8):
            m = bmbuf[pl.ds(i * 256, 16)]
            for j in range(1, 16):
                m = jnp.maximum(m, bmbuf[pl.ds(i * 256 + j * 16, 16)])
            bm2.append(m)

        R = (neg_splat, zero_i) * 4
        for i in range(8):
            R = _cascade_insert(R, bm2[i], zero_i)
        tau0 = R[6][15]
        tau0_splat = jnp.full((16,), tau0, jnp.float32)

        def ph_c(g, cnt):
            v = bmbuf[pl.ds(g * 16, 16)]
            m = v >= tau0_splat
            plsc.store_compressed(subbuf.at[pl.ds(cnt, 16)], iota + g * 16,
                                  mask=m)
            pc = plsc.all_reduce_population_count(m)
            return cnt + pc[0]
        nsub = lax.fori_loop(0, NGRP, ph_c, jnp.int32(0))

        def flush(R, cnt):
            nv = (cnt + 15) // 16
            def fl_body(q, R):
                base = q * 16
                kv = candv[pl.ds(base, 16)]
                vv = candi[pl.ds(base, 16)]
                valid = (iota + base) < cnt
                kv = jnp.where(valid, kv, neg_splat)
                gate = jnp.full((16,), R[6][15], jnp.float32)
                return _maybe_insert(R, kv, vv, gate)
            return lax.fori_loop(0, nv, fl_body, R)

        R = (neg_splat, zero_i) * 4

        def ph_d(j, carry):
            R = carry[:8]
            cnt = carry[8]
            sub = subbuf[pl.ds(j, 16)][0]
            base = (sub >> 4) * 256 + (sub & 15)
            idxv = base + iota * 16
            v = plsc.load_gather(databuf, [idxv])
            m = v >= tau0_splat
            plsc.store_compressed(candv.at[pl.ds(cnt, 16)], v, mask=m)
            plsc.store_compressed(candi.at[pl.ds(cnt, 16)], idxv, mask=m)
            pc = plsc.all_reduce_population_count(m)
            cnt = cnt + pc[0]
            R, cnt = lax.cond(
                cnt >= CAND_CAP - 16,
                lambda: (flush(R, cnt), jnp.int32(0)),
                lambda: (R, cnt))
            return (*R, cnt)

        carry = lax.fori_loop(0, nsub, ph_d, (*R, jnp.int32(0)))
        R = flush(carry[:8], carry[8])

        for i in range(4):
            stage_a[tt, pl.ds(i * 16, 16)] = R[2 * i]
            stage_i[tt, pl.ds(i * 16, 16)] = R[2 * i + 1]
        return 0

    lax.fori_loop(0, TPW, token_step, 0)
    pltpu.sync_copy(stage_a, acts_hbm.at[pl.ds(t0, TPW)])
    pltpu.sync_copy(stage_i, idx_hbm.at[pl.ds(t0, TPW)])


def _sc_topk(pre_acts):
    mesh = plsc.VectorSubcoreMesh(
        core_axis_name="c", subcore_axis_name="s",
        num_cores=NC, num_subcores=NS)
    fn = pl.kernel(
        _topk_body,
        out_type=(
            jax.ShapeDtypeStruct((T, K), jnp.float32),
            jax.ShapeDtypeStruct((T, K), jnp.int32),
        ),
        mesh=mesh,
        compiler_params=pltpu.CompilerParams(needs_layout_passes=False),
        scratch_types=[
            pltpu.VMEM((L,), jnp.float32),
            pltpu.VMEM((NSUB,), jnp.float32),
            pltpu.VMEM((NSUB + 16,), jnp.int32),
            pltpu.VMEM((CAND_CAP + 16,), jnp.float32),
            pltpu.VMEM((CAND_CAP + 16,), jnp.int32),
            pltpu.VMEM((TPW, K), jnp.float32),
            pltpu.VMEM((TPW, K), jnp.int32),
        ],
    )
    return fn(pre_acts)


def kernel(x, W_enc, b_enc, W_dec, b_dec):
    sae_in = x - b_dec[None, :]

    pre_acts = pl.pallas_call(
        _enc_body,
        grid=(L // _BL_ENC,),
        in_specs=[
            pl.BlockSpec((T, D), lambda j: (0, 0)),
            pl.BlockSpec((_BL_ENC, D), lambda j: (j, 0)),
            pl.BlockSpec((1, _BL_ENC), lambda j: (0, j)),
        ],
        out_specs=pl.BlockSpec((T, _BL_ENC), lambda j: (0, j)),
        out_shape=jax.ShapeDtypeStruct((T, L), jnp.float32),
    )(sae_in, W_enc, b_enc.reshape(1, L))

    top_acts, top_indices = _sc_topk(pre_acts)

    rows = jnp.arange(T)[:, None]
    z = jnp.zeros((T, L), dtype=x.dtype).at[rows, top_indices].add(top_acts)

    sae_out = pl.pallas_call(
        _dec_body,
        grid=(L // _BK_DEC,),
        in_specs=[
            pl.BlockSpec((T, _BK_DEC), lambda k: (0, k)),
            pl.BlockSpec((_BK_DEC, D), lambda k: (k, 0)),
            pl.BlockSpec((1, D), lambda k: (0, 0)),
        ],
        out_specs=pl.BlockSpec((T, D), lambda k: (0, 0)),
        out_shape=jax.ShapeDtypeStruct((T, D), jnp.float32),
    )(z, W_dec, b_dec.reshape(1, D))

    e = sae_out - x
    total_variance = jnp.sum((x - jnp.mean(x, axis=0)) ** 2)
    fvu = jnp.sum(e * e) / total_variance
    auxk_loss = jnp.zeros(())
    multi_topk_fvu = jnp.zeros(())
    return sae_out, top_acts, top_indices, fvu, auxk_loss, multi_topk_fvu
